# Initial kernel scaffold; baseline (speedup 1.0000x reference)
import jax
import jax.numpy as jnp
from jax.experimental import pallas as pl


def kernel(attn_bias, spatial_pos, edge_input, attn_edge_type, edge_enc_w, spatial_enc_w, edge_dis_w, vdist_w):
    raise NotImplementedError("write your pallas kernel here")


# speedup vs baseline: 12.5866x; 12.5866x over previous
import functools

import jax
import jax.numpy as jnp
from jax import lax
from jax.experimental import pallas as pl
from jax.experimental.pallas import tpu as pltpu
from jax.experimental.pallas import tpu_sc as plsc

_H = 32
_NUM_EDGES = 1536
_NUM_SPATIAL = 512
_MAX_DIST = 5
_CT_ROWS = _NUM_SPATIAL + _MAX_DIST * _NUM_EDGES
_NQ = 4
_HQ = _H // _NQ


def _tables_body(ew_ref, sw_ref, dis_ref, out_ref):
    ew = ew_ref[...]
    rid = lax.broadcasted_iota(jnp.int32, (ew.shape[0], 1), 0)
    ew = jnp.where(rid == 0, 0.0, ew)
    sw = sw_ref[...]
    rid2 = lax.broadcasted_iota(jnp.int32, (sw.shape[0], 1), 0)
    sw = jnp.where(rid2 == 0, 0.0, sw)
    parts = [sw]
    for d in range(_MAX_DIST):
        parts.append(jnp.dot(ew, dis_ref[d], preferred_element_type=jnp.float32))
    ct = jnp.concatenate(parts, axis=0)
    for q in range(_NQ):
        out_ref[q] = lax.slice(ct, (0, q * _HQ), (_CT_ROWS, (q + 1) * _HQ))


def _build_tables(edge_enc_w, spatial_enc_w, dis5):
    return pl.pallas_call(
        _tables_body,
        out_shape=jax.ShapeDtypeStruct((_NQ, _CT_ROWS, _HQ), jnp.float32),
    )(edge_enc_w, spatial_enc_w, dis5)


def _sc_interior(ct, eidx, sp, B, N):
    G = B * N
    NW = 32
    ROWS_PER_W = G // (NW // _NQ)

    mesh = plsc.VectorSubcoreMesh(core_axis_name="c", subcore_axis_name="s")

    @functools.partial(
        pl.kernel,
        mesh=mesh,
        out_type=jax.ShapeDtypeStruct((B, _H, N, N), jnp.float32),
        compiler_params=pltpu.CompilerParams(
            needs_layout_passes=False, use_tc_tiling_on_sc=False),
        scratch_types=[
            pltpu.VMEM((_CT_ROWS, _HQ), jnp.float32),
            pltpu.VMEM((15, N), jnp.int32),
            pltpu.VMEM((N,), jnp.int32),
            pltpu.VMEM((_HQ, 1, N), jnp.float32),
        ],
    )
    def sc_body(ct_hbm, eidx_hbm, sp_hbm, out_hbm, ct_v, eidx_v, sp_v, obuf):
        wid = lax.axis_index("s") * 2 + lax.axis_index("c")
        q = wid % _NQ
        rg = wid // _NQ

        pltpu.sync_copy(ct_hbm.at[q], ct_v)

        def row_body(r, carry):
            g = rg * ROWS_PER_W + r
            b = g // N
            i = g % N
            pltpu.sync_copy(eidx_hbm.at[g], eidx_v)
            pltpu.sync_copy(sp_hbm.at[g], sp_v)

            for g4 in range(N // 16):
                spv = sp_v[pl.ds(g4 * 16, 16)]
                sp1 = jnp.where(spv == 0, 1, spv)
                sp2 = jnp.where(sp1 > 1, sp1 - 1, sp1)
                sp3 = jnp.minimum(sp2, _MAX_DIST)
                rcpv = 1.0 / (3.0 * sp3.astype(jnp.float32))
                evs = [eidx_v[s, pl.ds(g4 * 16, 16)] for s in range(15)]
                for hq in range(_HQ):
                    hqv = jnp.full((16,), hq, jnp.int32)
                    sval = plsc.load_gather(ct_v, [spv, hqv])
                    eacc = plsc.load_gather(ct_v, [evs[0], hqv])
                    for s in range(1, 15):
                        eacc = eacc + plsc.load_gather(ct_v, [evs[s], hqv])
                    obuf[hq, 0, pl.ds(g4 * 16, 16)] = sval + rcpv * eacc

            pltpu.sync_copy(obuf, out_hbm.at[b, pl.ds(q * _HQ, _HQ), pl.ds(i, 1)])
            return carry

        lax.fori_loop(0, ROWS_PER_W, row_body, 0)

    return sc_body(ct, eidx, sp)


def _assemble_body(interim_ref, ab_ref, v_ref, out_ref):
    it = interim_ref[0]
    ab2 = ab_ref[0] * 2.0
    v = v_ref[0]
    out_ref[0, :, 1:, 1:] = it + ab2[1:, 1:][None, :, :]
    out_ref[0, :, 0:1, :] = (ab2[0, :][None, :] + v[:, None])[:, None, :]
    out_ref[0, :, 1:, 0:1] = (ab2[1:, 0][None, :] + v[:, None])[:, :, None]


def _assemble(interim, attn_bias, vdist_w, B, N):
    return pl.pallas_call(
        _assemble_body,
        grid=(B,),
        in_specs=[
            pl.BlockSpec((1, _H, N, N), lambda b: (b, 0, 0, 0)),
            pl.BlockSpec((1, N + 1, N + 1), lambda b: (b, 0, 0)),
            pl.BlockSpec((1, _H), lambda b: (0, 0)),
        ],
        out_specs=pl.BlockSpec((1, _H, N + 1, N + 1), lambda b: (b, 0, 0, 0)),
        out_shape=jax.ShapeDtypeStruct((B, _H, N + 1, N + 1), jnp.float32),
    )(interim, attn_bias, vdist_w)


def kernel(attn_bias, spatial_pos, edge_input, attn_edge_type,
           edge_enc_w, spatial_enc_w, edge_dis_w, vdist_w):
    del attn_edge_type
    B, N = spatial_pos.shape[:2]

    dis5 = edge_dis_w.reshape(-1, _H, _H)[:_MAX_DIST]
    ct = _build_tables(edge_enc_w, spatial_enc_w, dis5)

    ei = edge_input.astype(jnp.int32).transpose(0, 1, 3, 4, 2)
    off = (_NUM_SPATIAL + _NUM_EDGES * jnp.arange(_MAX_DIST, dtype=jnp.int32))
    eidx = (ei + off[None, None, :, None, None]).reshape(B * N, 15, N)
    sp = spatial_pos.astype(jnp.int32).reshape(B * N, N)

    interim = _sc_interior(ct, eidx, sp, B, N)
    return _assemble(interim, attn_bias, vdist_w, B, N)


# speedup vs baseline: 16.0009x; 1.2713x over previous
import functools

import jax
import jax.numpy as jnp
from jax import lax
from jax.experimental import pallas as pl
from jax.experimental.pallas import tpu as pltpu
from jax.experimental.pallas import tpu_sc as plsc

_H = 32
_NUM_EDGES = 1536
_NUM_SPATIAL = 512
_MAX_DIST = 5
_CT_ROWS = _NUM_SPATIAL + _MAX_DIST * _NUM_EDGES
_NQ = 4
_HQ = _H // _NQ


def _tables_body(ew_ref, sw_ref, dis_ref, out_ref):
    ew = ew_ref[...]
    rid = lax.broadcasted_iota(jnp.int32, (ew.shape[0], 1), 0)
    ew = jnp.where(rid == 0, 0.0, ew)
    sw = sw_ref[...]
    rid2 = lax.broadcasted_iota(jnp.int32, (sw.shape[0], 1), 0)
    sw = jnp.where(rid2 == 0, 0.0, sw)
    parts = [sw]
    for d in range(_MAX_DIST):
        parts.append(jnp.dot(ew, dis_ref[d], preferred_element_type=jnp.float32))
    ct = jnp.concatenate(parts, axis=0)
    for q in range(_NQ):
        out_ref[q] = lax.slice(ct, (0, q * _HQ), (_CT_ROWS, (q + 1) * _HQ))


def _build_tables(edge_enc_w, spatial_enc_w, dis5):
    return pl.pallas_call(
        _tables_body,
        out_shape=jax.ShapeDtypeStruct((_NQ, _CT_ROWS, _HQ), jnp.float32),
    )(edge_enc_w, spatial_enc_w, dis5)


_CH = 8


def _sc_interior(ct, idxall, B, N):
    G = B * N
    NW = 32
    ROWS_PER_W = G // (NW // _NQ)
    NCHUNK = ROWS_PER_W // _CH

    mesh = plsc.VectorSubcoreMesh(core_axis_name="c", subcore_axis_name="s")

    @functools.partial(
        pl.kernel,
        mesh=mesh,
        out_type=jax.ShapeDtypeStruct((B, _H, N, N), jnp.float32),
        compiler_params=pltpu.CompilerParams(
            needs_layout_passes=False, use_tc_tiling_on_sc=False),
        scratch_types=[
            pltpu.VMEM((_CT_ROWS, _HQ), jnp.float32),
            pltpu.VMEM((2, _CH, 16, N), jnp.int32),
            pltpu.VMEM((_HQ, _CH, N), jnp.float32),
            pltpu.SemaphoreType.DMA,
            pltpu.SemaphoreType.DMA,
        ],
    )
    def sc_body(ct_hbm, idx_hbm, out_hbm, ct_v, idx_v, obuf, sem0, sem1):
        wid = lax.axis_index("s") * 2 + lax.axis_index("c")
        q = wid % _NQ
        rg = wid // _NQ
        g_base = rg * ROWS_PER_W
        sems = (sem0, sem1)

        pltpu.sync_copy(ct_hbm.at[q], ct_v)
        pltpu.async_copy(idx_hbm.at[pl.ds(g_base, _CH)], idx_v.at[0], sem0)

        def compute_chunk(p, k):
            def row_body(rr, carry):
                for g4 in range(N // 16):
                    sl = pl.ds(g4 * 16, 16)
                    spv = idx_v[p, rr, 0, sl]
                    sp1 = jnp.where(spv == 0, 1, spv)
                    sp2 = jnp.where(sp1 > 1, sp1 - 1, sp1)
                    sp3 = jnp.minimum(sp2, _MAX_DIST)
                    rcpv = 1.0 / (3.0 * sp3.astype(jnp.float32))
                    evs = [idx_v[p, rr, 1 + s, sl] for s in range(15)]
                    for hq in range(_HQ):
                        hqv = jnp.full((16,), hq, jnp.int32)
                        sval = plsc.load_gather(ct_v, [spv, hqv])
                        c0 = plsc.load_gather(ct_v, [evs[0], hqv])
                        c1 = plsc.load_gather(ct_v, [evs[1], hqv])
                        c2 = plsc.load_gather(ct_v, [evs[2], hqv])
                        for s in range(3, 15, 3):
                            c0 = c0 + plsc.load_gather(ct_v, [evs[s], hqv])
                            c1 = c1 + plsc.load_gather(ct_v, [evs[s + 1], hqv])
                            c2 = c2 + plsc.load_gather(ct_v, [evs[s + 2], hqv])
                        eacc = (c0 + c1) + c2
                        obuf[hq, rr, sl] = sval + rcpv * eacc
                return carry

            lax.fori_loop(0, _CH, row_body, 0)
            g0 = g_base + k * _CH
            b = g0 // N
            i0 = g0 % N
            pltpu.sync_copy(obuf, out_hbm.at[b, pl.ds(q * _HQ, _HQ), pl.ds(i0, _CH)])

        def chunk_body(k2, carry):
            for par in range(2):
                k = k2 * 2 + par
                if par == 0:
                    pltpu.async_copy(
                        idx_hbm.at[pl.ds(g_base + (k + 1) * _CH, _CH)],
                        idx_v.at[1], sem1)
                else:
                    @pl.when(k2 < NCHUNK // 2 - 1)
                    def _():
                        pltpu.async_copy(
                            idx_hbm.at[pl.ds(g_base + (k + 1) * _CH, _CH)],
                            idx_v.at[0], sem0)
                pltpu.make_async_copy(
                    idx_hbm.at[pl.ds(g_base + k * _CH, _CH)],
                    idx_v.at[par], sems[par]).wait()
                compute_chunk(par, k)
            return carry

        lax.fori_loop(0, NCHUNK // 2, chunk_body, 0)

    return sc_body(ct, idxall)


def _assemble_body(interim_ref, ab_ref, v_ref, out_ref):
    it = interim_ref[0]
    ab2 = ab_ref[0] * 2.0
    v = v_ref[0]
    out_ref[0, :, 1:, 1:] = it + ab2[1:, 1:][None, :, :]
    out_ref[0, :, 0:1, :] = (ab2[0, :][None, :] + v[:, None])[:, None, :]
    out_ref[0, :, 1:, 0:1] = (ab2[1:, 0][None, :] + v[:, None])[:, :, None]


def _assemble(interim, attn_bias, vdist_w, B, N):
    return pl.pallas_call(
        _assemble_body,
        grid=(B,),
        in_specs=[
            pl.BlockSpec((1, _H, N, N), lambda b: (b, 0, 0, 0)),
            pl.BlockSpec((1, N + 1, N + 1), lambda b: (b, 0, 0)),
            pl.BlockSpec((1, _H), lambda b: (0, 0)),
        ],
        out_specs=pl.BlockSpec((1, _H, N + 1, N + 1), lambda b: (b, 0, 0, 0)),
        out_shape=jax.ShapeDtypeStruct((B, _H, N + 1, N + 1), jnp.float32),
    )(interim, attn_bias, vdist_w)


def kernel(attn_bias, spatial_pos, edge_input, attn_edge_type,
           edge_enc_w, spatial_enc_w, edge_dis_w, vdist_w):
    del attn_edge_type
    B, N = spatial_pos.shape[:2]

    dis5 = edge_dis_w.reshape(-1, _H, _H)[:_MAX_DIST]
    ct = _build_tables(edge_enc_w, spatial_enc_w, dis5)

    ei = edge_input.astype(jnp.int32).transpose(0, 1, 3, 4, 2)
    off = (_NUM_SPATIAL + _NUM_EDGES * jnp.arange(_MAX_DIST, dtype=jnp.int32))
    eidx = (ei + off[None, None, :, None, None]).reshape(B * N, 15, N)
    sp = spatial_pos.astype(jnp.int32).reshape(B * N, 1, N)
    idxall = jnp.concatenate([sp, eidx], axis=1)

    interim = _sc_interior(ct, idxall, B, N)
    return _assemble(interim, attn_bias, vdist_w, B, N)


# speedup vs baseline: 23.7573x; 1.4847x over previous
import functools

import jax
import jax.numpy as jnp
from jax import lax
from jax.experimental import pallas as pl
from jax.experimental.pallas import tpu as pltpu
from jax.experimental.pallas import tpu_sc as plsc

_H = 32
_NUM_EDGES = 1536
_NUM_SPATIAL = 512
_MAX_DIST = 5
_CT_ROWS = _NUM_SPATIAL + _MAX_DIST * _NUM_EDGES
_NQ = 4
_HQ = _H // _NQ
_CTS = 9


def _tables_body(ew_ref, sw_ref, dis_ref, out_ref):
    ew = ew_ref[...]
    rid = lax.broadcasted_iota(jnp.int32, (ew.shape[0], 1), 0)
    ew = jnp.where(rid == 0, 0.0, ew)
    sw = sw_ref[...]
    rid2 = lax.broadcasted_iota(jnp.int32, (sw.shape[0], 1), 0)
    sw = jnp.where(rid2 == 0, 0.0, sw)
    parts = [sw]
    for d in range(_MAX_DIST):
        parts.append(jnp.dot(ew, dis_ref[d], preferred_element_type=jnp.float32))
    ct = jnp.concatenate(parts, axis=0)
    pad = jnp.zeros((_CT_ROWS, _CTS - _HQ), jnp.float32)
    for q in range(_NQ):
        sl = lax.slice(ct, (0, q * _HQ), (_CT_ROWS, (q + 1) * _HQ))
        out_ref[q] = jnp.concatenate([sl, pad], axis=1)


def _build_tables(edge_enc_w, spatial_enc_w, dis5):
    return pl.pallas_call(
        _tables_body,
        out_shape=jax.ShapeDtypeStruct((_NQ, _CT_ROWS, _CTS), jnp.float32),
    )(edge_enc_w, spatial_enc_w, dis5)


_CH = 8


def _sc_interior(ct, idxall, B, N):
    G = B * N
    NW = 32
    ROWS_PER_W = G // (NW // _NQ)
    NCHUNK = ROWS_PER_W // _CH

    mesh = plsc.VectorSubcoreMesh(core_axis_name="c", subcore_axis_name="s")

    @functools.partial(
        pl.kernel,
        mesh=mesh,
        out_type=jax.ShapeDtypeStruct((B, _H, N, N), jnp.float32),
        compiler_params=pltpu.CompilerParams(
            needs_layout_passes=False, use_tc_tiling_on_sc=False),
        scratch_types=[
            pltpu.VMEM((_CT_ROWS * _CTS,), jnp.float32),
            pltpu.VMEM((2, _CH, 16, N), jnp.int32),
            pltpu.VMEM((2, _HQ, _CH, N), jnp.float32),
            pltpu.SemaphoreType.DMA,
            pltpu.SemaphoreType.DMA,
            pltpu.SemaphoreType.DMA,
            pltpu.SemaphoreType.DMA,
        ],
    )
    def sc_body(ct_hbm, idx_hbm, out_hbm, ct_v, idx_v, obuf, sem0, sem1,
                osem0, osem1):
        wid = lax.axis_index("s") * 2 + lax.axis_index("c")
        q = wid % _NQ
        rg = wid // _NQ
        g_base = rg * ROWS_PER_W
        sems = (sem0, sem1)
        osems = (osem0, osem1)

        pltpu.sync_copy(ct_hbm.at[q], ct_v)
        pltpu.async_copy(idx_hbm.at[pl.ds(g_base, _CH)], idx_v.at[0], sem0)

        def out_slice(k):
            g0 = g_base + k * _CH
            return out_hbm.at[g0 // N, pl.ds(q * _HQ, _HQ), pl.ds(g0 % N, _CH)]

        def compute_chunk(p, k):
            def row_body(rr, carry):
                for g4 in range(N // 16):
                    sl = pl.ds(g4 * 16, 16)
                    spv = idx_v[p, rr, 0, sl]
                    sp1 = jnp.where(spv == 0, 1, spv)
                    sp2 = jnp.where(sp1 > 1, sp1 - 1, sp1)
                    sp3 = jnp.minimum(sp2, _MAX_DIST)
                    rcpv = 1.0 / (3.0 * sp3.astype(jnp.float32))
                    spv9 = spv * _CTS
                    evs9 = [idx_v[p, rr, 1 + s, sl] * _CTS for s in range(15)]
                    for hq in range(_HQ):
                        hqv = jnp.full((16,), hq, jnp.int32)
                        sval = plsc.load_gather(ct_v, [spv9 + hqv])
                        c0 = plsc.load_gather(ct_v, [evs9[0] + hqv])
                        c1 = plsc.load_gather(ct_v, [evs9[1] + hqv])
                        c2 = plsc.load_gather(ct_v, [evs9[2] + hqv])
                        for s in range(3, 15, 3):
                            c0 = c0 + plsc.load_gather(ct_v, [evs9[s] + hqv])
                            c1 = c1 + plsc.load_gather(ct_v, [evs9[s + 1] + hqv])
                            c2 = c2 + plsc.load_gather(ct_v, [evs9[s + 2] + hqv])
                        eacc = (c0 + c1) + c2
                        obuf[p, hq, rr, sl] = sval + rcpv * eacc
                return carry

            lax.fori_loop(0, _CH, row_body, 0)
            pltpu.async_copy(obuf.at[p], out_slice(k), osems[p])

        def chunk_body(k2, carry):
            for par in range(2):
                k = k2 * 2 + par
                if par == 0:
                    pltpu.async_copy(
                        idx_hbm.at[pl.ds(g_base + (k + 1) * _CH, _CH)],
                        idx_v.at[1], sem1)
                else:
                    @pl.when(k2 < NCHUNK // 2 - 1)
                    def _():
                        pltpu.async_copy(
                            idx_hbm.at[pl.ds(g_base + (k + 1) * _CH, _CH)],
                            idx_v.at[0], sem0)
                pltpu.make_async_copy(
                    idx_hbm.at[pl.ds(g_base + k * _CH, _CH)],
                    idx_v.at[par], sems[par]).wait()
                @pl.when(k >= 2)
                def _():
                    pltpu.make_async_copy(
                        obuf.at[par], out_slice(k - 2), osems[par]).wait()
                compute_chunk(par, k)
            return carry

        lax.fori_loop(0, NCHUNK // 2, chunk_body, 0)
        pltpu.make_async_copy(obuf.at[0], out_slice(NCHUNK - 2), osem0).wait()
        pltpu.make_async_copy(obuf.at[1], out_slice(NCHUNK - 1), osem1).wait()

    return sc_body(ct, idxall)


def _assemble_body(interim_ref, ab_ref, v_ref, out_ref):
    it = interim_ref[0]
    ab2 = ab_ref[0] * 2.0
    v = v_ref[0]
    out_ref[0, :, 1:, 1:] = it + ab2[1:, 1:][None, :, :]
    out_ref[0, :, 0:1, :] = (ab2[0, :][None, :] + v[:, None])[:, None, :]
    out_ref[0, :, 1:, 0:1] = (ab2[1:, 0][None, :] + v[:, None])[:, :, None]


def _assemble(interim, attn_bias, vdist_w, B, N):
    return pl.pallas_call(
        _assemble_body,
        grid=(B,),
        in_specs=[
            pl.BlockSpec((1, _H, N, N), lambda b: (b, 0, 0, 0)),
            pl.BlockSpec((1, N + 1, N + 1), lambda b: (b, 0, 0)),
            pl.BlockSpec((1, _H), lambda b: (0, 0)),
        ],
        out_specs=pl.BlockSpec((1, _H, N + 1, N + 1), lambda b: (b, 0, 0, 0)),
        out_shape=jax.ShapeDtypeStruct((B, _H, N + 1, N + 1), jnp.float32),
    )(interim, attn_bias, vdist_w)


def kernel(attn_bias, spatial_pos, edge_input, attn_edge_type,
           edge_enc_w, spatial_enc_w, edge_dis_w, vdist_w):
    del attn_edge_type
    B, N = spatial_pos.shape[:2]

    dis5 = edge_dis_w.reshape(-1, _H, _H)[:_MAX_DIST]
    ct = _build_tables(edge_enc_w, spatial_enc_w, dis5)
    ct = ct.reshape(_NQ, _CT_ROWS * _CTS)

    ei = edge_input.astype(jnp.int32).transpose(0, 1, 3, 4, 2)
    off = (_NUM_SPATIAL + _NUM_EDGES * jnp.arange(_MAX_DIST, dtype=jnp.int32))
    eidx = (ei + off[None, None, :, None, None]).reshape(B * N, 15, N)
    sp = spatial_pos.astype(jnp.int32).reshape(B * N, 1, N)
    idxall = jnp.concatenate([sp, eidx], axis=1)

    interim = _sc_interior(ct, idxall, B, N)
    return _assemble(interim, attn_bias, vdist_w, B, N)


# speedup vs baseline: 25.9478x; 1.0922x over previous
import functools

import jax
import jax.numpy as jnp
from jax import lax
from jax.experimental import pallas as pl
from jax.experimental.pallas import tpu as pltpu
from jax.experimental.pallas import tpu_sc as plsc

_H = 32
_NUM_EDGES = 1536
_NUM_SPATIAL = 512
_MAX_DIST = 5
_CT_ROWS = _NUM_SPATIAL + _MAX_DIST * _NUM_EDGES
_NQ = 4
_HQ = _H // _NQ
_CTS = 9


def _tables_body(ew_ref, sw_ref, dis_ref, out_ref):
    ew = ew_ref[...]
    rid = lax.broadcasted_iota(jnp.int32, (ew.shape[0], 1), 0)
    ew = jnp.where(rid == 0, 0.0, ew)
    sw = sw_ref[...]
    rid2 = lax.broadcasted_iota(jnp.int32, (sw.shape[0], 1), 0)
    sw = jnp.where(rid2 == 0, 0.0, sw)
    parts = [sw]
    for d in range(_MAX_DIST):
        parts.append(jnp.dot(ew, dis_ref[d], preferred_element_type=jnp.float32))
    ct = jnp.concatenate(parts, axis=0)
    pad = jnp.zeros((_CT_ROWS, _CTS - _HQ), jnp.float32)
    for q in range(_NQ):
        sl = lax.slice(ct, (0, q * _HQ), (_CT_ROWS, (q + 1) * _HQ))
        out_ref[q] = jnp.concatenate([sl, pad], axis=1)


def _build_tables(edge_enc_w, spatial_enc_w, dis5):
    return pl.pallas_call(
        _tables_body,
        out_shape=jax.ShapeDtypeStruct((_NQ, _CT_ROWS, _CTS), jnp.float32),
    )(edge_enc_w, spatial_enc_w, dis5)


_CH = 8


def _sc_interior(ct, sp, ei, B, N):
    G = B * N
    NW = 32
    ROWS_PER_W = G // (NW // _NQ)
    NCHUNK = ROWS_PER_W // _CH

    mesh = plsc.VectorSubcoreMesh(core_axis_name="c", subcore_axis_name="s")

    @functools.partial(
        pl.kernel,
        mesh=mesh,
        out_type=jax.ShapeDtypeStruct((B, _H, N, N), jnp.float32),
        compiler_params=pltpu.CompilerParams(
            needs_layout_passes=False, use_tc_tiling_on_sc=False),
        scratch_types=[
            pltpu.VMEM((_CT_ROWS * _CTS,), jnp.float32),
            pltpu.VMEM((2, _CH, N), jnp.int32),
            pltpu.VMEM((2 * _CH, 15 * N), jnp.int32),
            pltpu.VMEM((2, _HQ, _CH, N), jnp.float32),
            pltpu.SemaphoreType.DMA,
            pltpu.SemaphoreType.DMA,
            pltpu.SemaphoreType.DMA,
            pltpu.SemaphoreType.DMA,
        ],
    )
    def sc_body(ct_hbm, sp_hbm, ei_hbm, out_hbm, ct_v, sp_v, ei_v, obuf,
                sem0, sem1, osem0, osem1):
        wid = lax.axis_index("s") * 2 + lax.axis_index("c")
        q = wid % _NQ
        rg = wid // _NQ
        g_base = rg * ROWS_PER_W
        sems = (sem0, sem1)
        osems = (osem0, osem1)

        pltpu.sync_copy(ct_hbm.at[q], ct_v)

        def fetch(k, p, sem):
            g0 = g_base + k * _CH
            pltpu.async_copy(sp_hbm.at[pl.ds(g0, _CH)], sp_v.at[p], sem)
            pltpu.async_copy(ei_hbm.at[pl.ds(g0, _CH)],
                             ei_v.at[pl.ds(p * _CH, _CH)], sem)

        def wait_fetch(k, p, sem):
            g0 = g_base + k * _CH
            pltpu.make_async_copy(sp_hbm.at[pl.ds(g0, _CH)], sp_v.at[p], sem).wait()
            pltpu.make_async_copy(ei_hbm.at[pl.ds(g0, _CH)],
                                  ei_v.at[pl.ds(p * _CH, _CH)], sem).wait()

        fetch(0, 0, sem0)

        def out_slice(k):
            g0 = g_base + k * _CH
            return out_hbm.at[g0 // N, pl.ds(q * _HQ, _HQ), pl.ds(g0 % N, _CH)]

        lane15 = jnp.arange(16, dtype=jnp.int32) * 15
        offs = [
            jnp.full((16,), (_NUM_SPATIAL + _NUM_EDGES * (s // 3)) * _CTS,
                     jnp.int32)
            for s in range(15)
        ]

        def compute_chunk(p, k):
            def row_body(rr, carry):
                rowv = jnp.full((16,), p * _CH, jnp.int32) + rr
                for g4 in range(N // 16):
                    sl = pl.ds(g4 * 16, 16)
                    spv = sp_v[p, rr, sl]
                    sp1 = jnp.where(spv == 0, 1, spv)
                    sp2 = jnp.where(sp1 > 1, sp1 - 1, sp1)
                    sp3 = jnp.minimum(sp2, _MAX_DIST)
                    rcpv = 1.0 / (3.0 * sp3.astype(jnp.float32))
                    spv9 = spv * _CTS
                    evs9 = [
                        plsc.load_gather(ei_v, [rowv, lane15 + (g4 * 240 + s)])
                        * _CTS + offs[s]
                        for s in range(15)
                    ]
                    for hq in range(_HQ):
                        hqv = jnp.full((16,), hq, jnp.int32)
                        sval = plsc.load_gather(ct_v, [spv9 + hqv])
                        c0 = plsc.load_gather(ct_v, [evs9[0] + hqv])
                        c1 = plsc.load_gather(ct_v, [evs9[1] + hqv])
                        c2 = plsc.load_gather(ct_v, [evs9[2] + hqv])
                        for s in range(3, 15, 3):
                            c0 = c0 + plsc.load_gather(ct_v, [evs9[s] + hqv])
                            c1 = c1 + plsc.load_gather(ct_v, [evs9[s + 1] + hqv])
                            c2 = c2 + plsc.load_gather(ct_v, [evs9[s + 2] + hqv])
                        eacc = (c0 + c1) + c2
                        obuf[p, hq, rr, sl] = sval + rcpv * eacc
                return carry

            lax.fori_loop(0, _CH, row_body, 0)
            pltpu.async_copy(obuf.at[p], out_slice(k), osems[p])

        def chunk_body(k2, carry):
            for par in range(2):
                k = k2 * 2 + par
                if par == 0:
                    fetch(k + 1, 1, sem1)
                else:
                    @pl.when(k2 < NCHUNK // 2 - 1)
                    def _():
                        fetch(k + 1, 0, sem0)
                wait_fetch(k, par, sems[par])
                @pl.when(k >= 2)
                def _():
                    pltpu.make_async_copy(
                        obuf.at[par], out_slice(k - 2), osems[par]).wait()
                compute_chunk(par, k)
            return carry

        lax.fori_loop(0, NCHUNK // 2, chunk_body, 0)
        pltpu.make_async_copy(obuf.at[0], out_slice(NCHUNK - 2), osem0).wait()
        pltpu.make_async_copy(obuf.at[1], out_slice(NCHUNK - 1), osem1).wait()

    return sc_body(ct, sp, ei)


def _assemble_body(interim_ref, ab_ref, v_ref, out_ref):
    it = interim_ref[0]
    ab2 = ab_ref[0] * 2.0
    v = v_ref[0]
    out_ref[0, :, 1:, 1:] = it + ab2[1:, 1:][None, :, :]
    out_ref[0, :, 0:1, :] = (ab2[0, :][None, :] + v[:, None])[:, None, :]
    out_ref[0, :, 1:, 0:1] = (ab2[1:, 0][None, :] + v[:, None])[:, :, None]


def _assemble(interim, attn_bias, vdist_w, B, N):
    return pl.pallas_call(
        _assemble_body,
        grid=(B,),
        in_specs=[
            pl.BlockSpec((1, _H, N, N), lambda b: (b, 0, 0, 0)),
            pl.BlockSpec((1, N + 1, N + 1), lambda b: (b, 0, 0)),
            pl.BlockSpec((1, _H), lambda b: (0, 0)),
        ],
        out_specs=pl.BlockSpec((1, _H, N + 1, N + 1), lambda b: (b, 0, 0, 0)),
        out_shape=jax.ShapeDtypeStruct((B, _H, N + 1, N + 1), jnp.float32),
    )(interim, attn_bias, vdist_w)


def kernel(attn_bias, spatial_pos, edge_input, attn_edge_type,
           edge_enc_w, spatial_enc_w, edge_dis_w, vdist_w):
    del attn_edge_type
    B, N = spatial_pos.shape[:2]

    dis5 = edge_dis_w.reshape(-1, _H, _H)[:_MAX_DIST]
    ct = _build_tables(edge_enc_w, spatial_enc_w, dis5)
    ct = ct.reshape(_NQ, _CT_ROWS * _CTS)

    sp = spatial_pos.astype(jnp.int32).reshape(B * N, N)
    ei = edge_input.astype(jnp.int32).reshape(B * N, N * 15)

    interim = _sc_interior(ct, sp, ei, B, N)
    return _assemble(interim, attn_bias, vdist_w, B, N)


# speedup vs baseline: 26.4379x; 1.0189x over previous
import functools

import jax
import jax.numpy as jnp
from jax import lax
from jax.experimental import pallas as pl
from jax.experimental.pallas import tpu as pltpu
from jax.experimental.pallas import tpu_sc as plsc

_H = 32
_NUM_EDGES = 1536
_NUM_SPATIAL = 512
_MAX_DIST = 5
_CT_ROWS = _NUM_SPATIAL + _MAX_DIST * _NUM_EDGES
_NQ = 4
_HQ = _H // _NQ
_CTS = 5


def _tables_body(ew_ref, sw_ref, dis_ref, out_ref):
    ew = ew_ref[...]
    rid = lax.broadcasted_iota(jnp.int32, (ew.shape[0], 1), 0)
    ew = jnp.where(rid == 0, 0.0, ew)
    sw = sw_ref[...]
    rid2 = lax.broadcasted_iota(jnp.int32, (sw.shape[0], 1), 0)
    sw = jnp.where(rid2 == 0, 0.0, sw)
    parts = [sw]
    for d in range(_MAX_DIST):
        parts.append(jnp.dot(ew, dis_ref[d], preferred_element_type=jnp.float32))
    ct = jnp.concatenate(parts, axis=0)
    for q in range(_NQ):
        out_ref[q] = lax.slice(ct, (0, q * _HQ), (_CT_ROWS, (q + 1) * _HQ))


def _build_tables(edge_enc_w, spatial_enc_w, dis5):
    return pl.pallas_call(
        _tables_body,
        out_shape=jax.ShapeDtypeStruct((_NQ, _CT_ROWS, _HQ), jnp.float32),
    )(edge_enc_w, spatial_enc_w, dis5)


_CH = 8


def _sc_interior(ct, sp, ei, B, N):
    G = B * N
    NW = 32
    ROWS_PER_W = G // (NW // _NQ)
    NCHUNK = ROWS_PER_W // _CH

    mesh = plsc.VectorSubcoreMesh(core_axis_name="c", subcore_axis_name="s")

    @functools.partial(
        pl.kernel,
        mesh=mesh,
        out_type=jax.ShapeDtypeStruct((B, _H, N, N), jnp.float32),
        compiler_params=pltpu.CompilerParams(
            needs_layout_passes=False, use_tc_tiling_on_sc=False),
        scratch_types=[
            pltpu.VMEM((_CT_ROWS * _CTS,), jnp.int32),
            pltpu.VMEM((2, _CH, N), jnp.int32),
            pltpu.VMEM((2 * _CH, 15 * N), jnp.int32),
            pltpu.VMEM((2, _HQ, _CH, N), jnp.float32),
            pltpu.SemaphoreType.DMA,
            pltpu.SemaphoreType.DMA,
            pltpu.SemaphoreType.DMA,
            pltpu.SemaphoreType.DMA,
        ],
    )
    def sc_body(ct_hbm, sp_hbm, ei_hbm, out_hbm, ct_v, sp_v, ei_v, obuf,
                sem0, sem1, osem0, osem1):
        wid = lax.axis_index("s") * 2 + lax.axis_index("c")
        q = wid % _NQ
        rg = wid // _NQ
        g_base = rg * ROWS_PER_W
        sems = (sem0, sem1)
        osems = (osem0, osem1)

        pltpu.sync_copy(ct_hbm.at[q], ct_v)

        def fetch(k, p, sem):
            g0 = g_base + k * _CH
            pltpu.async_copy(sp_hbm.at[pl.ds(g0, _CH)], sp_v.at[p], sem)
            pltpu.async_copy(ei_hbm.at[pl.ds(g0, _CH)],
                             ei_v.at[pl.ds(p * _CH, _CH)], sem)

        def wait_fetch(k, p, sem):
            g0 = g_base + k * _CH
            pltpu.make_async_copy(sp_hbm.at[pl.ds(g0, _CH)], sp_v.at[p], sem).wait()
            pltpu.make_async_copy(ei_hbm.at[pl.ds(g0, _CH)],
                                  ei_v.at[pl.ds(p * _CH, _CH)], sem).wait()

        fetch(0, 0, sem0)

        def out_slice(k):
            g0 = g_base + k * _CH
            return out_hbm.at[g0 // N, pl.ds(q * _HQ, _HQ), pl.ds(g0 % N, _CH)]

        lane15 = jnp.arange(16, dtype=jnp.int32) * 15
        offs = [
            jnp.full((16,), (_NUM_SPATIAL + _NUM_EDGES * (s // 3)) * _CTS,
                     jnp.int32)
            for s in range(15)
        ]

        def compute_chunk(p, k):
            def row_body(rr, carry):
                rowv = jnp.full((16,), p * _CH, jnp.int32) + rr
                for g4 in range(N // 16):
                    sl = pl.ds(g4 * 16, 16)
                    spv = sp_v[p, rr, sl]
                    sp1 = jnp.where(spv == 0, 1, spv)
                    sp2 = jnp.where(sp1 > 1, sp1 - 1, sp1)
                    sp3 = jnp.minimum(sp2, _MAX_DIST)
                    rcpv = 1.0 / (3.0 * sp3.astype(jnp.float32))
                    rcp_bf = plsc.pack(rcpv, rcpv, format=plsc.PackFormat.INTERLEAVED)
                    spv5 = spv * _CTS
                    evs5 = [
                        plsc.load_gather(ei_v, [rowv, lane15 + (g4 * 240 + s)])
                        * _CTS + offs[s]
                        for s in range(15)
                    ]

                    def gat(base, wqv):
                        w = plsc.load_gather(ct_v, [base + wqv])
                        return plsc.bitcast(w, jnp.bfloat16)

                    for wq in range(_HQ // 2):
                        wqv = jnp.full((16,), wq, jnp.int32)
                        sval = gat(spv5, wqv)
                        c0 = gat(evs5[0], wqv)
                        c1 = gat(evs5[1], wqv)
                        c2 = gat(evs5[2], wqv)
                        for s in range(3, 15, 3):
                            c0 = c0 + gat(evs5[s], wqv)
                            c1 = c1 + gat(evs5[s + 1], wqv)
                            c2 = c2 + gat(evs5[s + 2], wqv)
                        outv = sval + rcp_bf * ((c0 + c1) + c2)
                        lo, hi = plsc.unpack(outv, format=plsc.PackFormat.INTERLEAVED)
                        obuf[p, 2 * wq, rr, sl] = lo
                        obuf[p, 2 * wq + 1, rr, sl] = hi
                return carry

            lax.fori_loop(0, _CH, row_body, 0)
            pltpu.async_copy(obuf.at[p], out_slice(k), osems[p])

        def chunk_body(k2, carry):
            for par in range(2):
                k = k2 * 2 + par
                if par == 0:
                    fetch(k + 1, 1, sem1)
                else:
                    @pl.when(k2 < NCHUNK // 2 - 1)
                    def _():
                        fetch(k + 1, 0, sem0)
                wait_fetch(k, par, sems[par])
                @pl.when(k >= 2)
                def _():
                    pltpu.make_async_copy(
                        obuf.at[par], out_slice(k - 2), osems[par]).wait()
                compute_chunk(par, k)
            return carry

        lax.fori_loop(0, NCHUNK // 2, chunk_body, 0)
        pltpu.make_async_copy(obuf.at[0], out_slice(NCHUNK - 2), osem0).wait()
        pltpu.make_async_copy(obuf.at[1], out_slice(NCHUNK - 1), osem1).wait()

    return sc_body(ct, sp, ei)


def _assemble_body(interim_ref, ab_ref, v_ref, out_ref):
    it = interim_ref[0]
    ab2 = ab_ref[0] * 2.0
    v = v_ref[0]
    out_ref[0, :, 1:, 1:] = it + ab2[1:, 1:][None, :, :]
    out_ref[0, :, 0:1, :] = (ab2[0, :][None, :] + v[:, None])[:, None, :]
    out_ref[0, :, 1:, 0:1] = (ab2[1:, 0][None, :] + v[:, None])[:, :, None]


def _assemble(interim, attn_bias, vdist_w, B, N):
    return pl.pallas_call(
        _assemble_body,
        grid=(B,),
        in_specs=[
            pl.BlockSpec((1, _H, N, N), lambda b: (b, 0, 0, 0)),
            pl.BlockSpec((1, N + 1, N + 1), lambda b: (b, 0, 0)),
            pl.BlockSpec((1, _H), lambda b: (0, 0)),
        ],
        out_specs=pl.BlockSpec((1, _H, N + 1, N + 1), lambda b: (b, 0, 0, 0)),
        out_shape=jax.ShapeDtypeStruct((B, _H, N + 1, N + 1), jnp.float32),
    )(interim, attn_bias, vdist_w)


def kernel(attn_bias, spatial_pos, edge_input, attn_edge_type,
           edge_enc_w, spatial_enc_w, edge_dis_w, vdist_w):
    del attn_edge_type
    B, N = spatial_pos.shape[:2]

    dis5 = edge_dis_w.reshape(-1, _H, _H)[:_MAX_DIST]
    ct = _build_tables(edge_enc_w, spatial_enc_w, dis5)
    ctb = ct.astype(jnp.bfloat16).reshape(_NQ, _CT_ROWS, _HQ // 2, 2)
    ctw = jax.lax.bitcast_convert_type(ctb, jnp.int32)
    ctw = jnp.concatenate(
        [ctw, jnp.zeros((_NQ, _CT_ROWS, 1), jnp.int32)], axis=2)
    ct = ctw.reshape(_NQ, _CT_ROWS * _CTS)

    sp = spatial_pos.astype(jnp.int32).reshape(B * N, N)
    ei = edge_input.astype(jnp.int32).reshape(B * N, N * 15)

    interim = _sc_interior(ct, sp, ei, B, N)
    return _assemble(interim, attn_bias, vdist_w, B, N)


# speedup vs baseline: 32.1137x; 1.2147x over previous
import functools

import jax
import jax.numpy as jnp
from jax import lax
from jax.experimental import pallas as pl
from jax.experimental.pallas import tpu as pltpu
from jax.experimental.pallas import tpu_sc as plsc

_H = 32
_NUM_EDGES = 1536
_NUM_SPATIAL = 512
_MAX_DIST = 5
_CT_ROWS = _NUM_SPATIAL + _MAX_DIST * _NUM_EDGES
_NQ = 4
_HQ = _H // _NQ
_CTS = 5


def _tables_body(ew_ref, sw_ref, dis_ref, out_ref):
    ew = ew_ref[...]
    rid = lax.broadcasted_iota(jnp.int32, (ew.shape[0], 1), 0)
    ew = jnp.where(rid == 0, 0.0, ew)
    sw = sw_ref[...]
    rid2 = lax.broadcasted_iota(jnp.int32, (sw.shape[0], 1), 0)
    sw = jnp.where(rid2 == 0, 0.0, sw)
    parts = [sw]
    for d in range(_MAX_DIST):
        parts.append(jnp.dot(ew, dis_ref[d], preferred_element_type=jnp.float32))
    ct = jnp.concatenate(parts, axis=0)
    xi = lax.bitcast_convert_type(ct, jnp.int32)
    r16 = ((xi + 0x7FFF + ((xi >> 16) & 1)) >> 16) & 0xFFFF
    zcol = jnp.zeros((_CT_ROWS, 1), jnp.int32)
    for q in range(_NQ):
        lo = lax.slice(r16, (0, q * _HQ), (_CT_ROWS, q * _HQ + 4))
        hi = lax.slice(r16, (0, q * _HQ + 4), (_CT_ROWS, (q + 1) * _HQ))
        out_ref[q] = jnp.concatenate([lo | (hi << 16), zcol], axis=1)


def _build_tables(edge_enc_w, spatial_enc_w, dis5):
    return pl.pallas_call(
        _tables_body,
        out_shape=jax.ShapeDtypeStruct((_NQ, _CT_ROWS, _CTS), jnp.int32),
    )(edge_enc_w, spatial_enc_w, dis5)


_CH = 8


def _sc_interior(ct, sp, ei, B, N):
    G = B * N
    NW = 32
    ROWS_PER_W = G // (NW // _NQ)
    NCHUNK = ROWS_PER_W // _CH

    mesh = plsc.VectorSubcoreMesh(core_axis_name="c", subcore_axis_name="s")

    @functools.partial(
        pl.kernel,
        mesh=mesh,
        out_type=jax.ShapeDtypeStruct((B, _H, N, N), jnp.float32),
        compiler_params=pltpu.CompilerParams(
            needs_layout_passes=False, use_tc_tiling_on_sc=False),
        scratch_types=[
            pltpu.VMEM((_CT_ROWS * _CTS,), jnp.int32),
            pltpu.VMEM((2, _CH, N), jnp.int32),
            pltpu.VMEM((2 * _CH, 15 * N), jnp.int32),
            pltpu.VMEM((2, _HQ, _CH, N), jnp.float32),
            pltpu.SemaphoreType.DMA,
            pltpu.SemaphoreType.DMA,
            pltpu.SemaphoreType.DMA,
            pltpu.SemaphoreType.DMA,
        ],
    )
    def sc_body(ct_hbm, sp_hbm, ei_hbm, out_hbm, ct_v, sp_v, ei_v, obuf,
                sem0, sem1, osem0, osem1):
        wid = lax.axis_index("s") * 2 + lax.axis_index("c")
        q = wid % _NQ
        rg = wid // _NQ
        g_base = rg * ROWS_PER_W
        sems = (sem0, sem1)
        osems = (osem0, osem1)

        pltpu.sync_copy(ct_hbm.at[q], ct_v)

        def fetch(k, p, sem):
            g0 = g_base + k * _CH
            pltpu.async_copy(sp_hbm.at[pl.ds(g0, _CH)], sp_v.at[p], sem)
            pltpu.async_copy(ei_hbm.at[pl.ds(g0, _CH)],
                             ei_v.at[pl.ds(p * _CH, _CH)], sem)

        def wait_fetch(k, p, sem):
            g0 = g_base + k * _CH
            pltpu.make_async_copy(sp_hbm.at[pl.ds(g0, _CH)], sp_v.at[p], sem).wait()
            pltpu.make_async_copy(ei_hbm.at[pl.ds(g0, _CH)],
                                  ei_v.at[pl.ds(p * _CH, _CH)], sem).wait()

        fetch(0, 0, sem0)

        def out_slice(k):
            g0 = g_base + k * _CH
            return out_hbm.at[g0 // N, pl.ds(q * _HQ, _HQ), pl.ds(g0 % N, _CH)]

        lane15 = jnp.arange(16, dtype=jnp.int32) * 15
        offs = [
            jnp.full((16,), (_NUM_SPATIAL + _NUM_EDGES * (s // 3)) * _CTS,
                     jnp.int32)
            for s in range(15)
        ]

        def compute_chunk(p, k):
            def row_body(rr, carry):
                rowv = jnp.full((16,), p * _CH, jnp.int32) + rr
                for g4 in range(N // 16):
                    sl = pl.ds(g4 * 16, 16)
                    spv = sp_v[p, rr, sl]
                    sp1 = jnp.where(spv == 0, 1, spv)
                    sp2 = jnp.where(sp1 > 1, sp1 - 1, sp1)
                    sp3 = jnp.minimum(sp2, _MAX_DIST)
                    rcpv = 1.0 / (3.0 * sp3.astype(jnp.float32))
                    rcp_bf = plsc.pack(rcpv, rcpv, format=plsc.PackFormat.INTERLEAVED)
                    spv5 = spv * _CTS
                    evs5 = [
                        plsc.load_gather(ei_v, [rowv, lane15 + (g4 * 240 + s)])
                        * _CTS + offs[s]
                        for s in range(15)
                    ]

                    def gat(base, wqv):
                        w = plsc.load_gather(ct_v, [base + wqv])
                        return plsc.bitcast(w, jnp.bfloat16)

                    for wq in range(_HQ // 2):
                        wqv = jnp.full((16,), wq, jnp.int32)
                        sval = gat(spv5, wqv)
                        c0 = gat(evs5[0], wqv)
                        c1 = gat(evs5[1], wqv)
                        c2 = gat(evs5[2], wqv)
                        for s in range(3, 15, 3):
                            c0 = c0 + gat(evs5[s], wqv)
                            c1 = c1 + gat(evs5[s + 1], wqv)
                            c2 = c2 + gat(evs5[s + 2], wqv)
                        outv = sval + rcp_bf * ((c0 + c1) + c2)
                        lo, hi = plsc.unpack(outv, format=plsc.PackFormat.INTERLEAVED)
                        obuf[p, wq, rr, sl] = lo
                        obuf[p, wq + 4, rr, sl] = hi
                return carry

            lax.fori_loop(0, _CH, row_body, 0)
            pltpu.async_copy(obuf.at[p], out_slice(k), osems[p])

        def chunk_body(k2, carry):
            for par in range(2):
                k = k2 * 2 + par
                if par == 0:
                    fetch(k + 1, 1, sem1)
                else:
                    @pl.when(k2 < NCHUNK // 2 - 1)
                    def _():
                        fetch(k + 1, 0, sem0)
                wait_fetch(k, par, sems[par])
                @pl.when(k >= 2)
                def _():
                    pltpu.make_async_copy(
                        obuf.at[par], out_slice(k - 2), osems[par]).wait()
                compute_chunk(par, k)
            return carry

        lax.fori_loop(0, NCHUNK // 2, chunk_body, 0)
        pltpu.make_async_copy(obuf.at[0], out_slice(NCHUNK - 2), osem0).wait()
        pltpu.make_async_copy(obuf.at[1], out_slice(NCHUNK - 1), osem1).wait()

    return sc_body(ct, sp, ei)


def _assemble_body(interim_ref, ab_ref, v_ref, out_ref):
    it = interim_ref[0]
    ab2 = ab_ref[0] * 2.0
    v = v_ref[0]
    out_ref[0, :, 1:, 1:] = it + ab2[1:, 1:][None, :, :]
    out_ref[0, :, 0:1, :] = (ab2[0, :][None, :] + v[:, None])[:, None, :]
    out_ref[0, :, 1:, 0:1] = (ab2[1:, 0][None, :] + v[:, None])[:, :, None]


def _assemble(interim, attn_bias, vdist_w, B, N):
    return pl.pallas_call(
        _assemble_body,
        grid=(B,),
        in_specs=[
            pl.BlockSpec((1, _H, N, N), lambda b: (b, 0, 0, 0)),
            pl.BlockSpec((1, N + 1, N + 1), lambda b: (b, 0, 0)),
            pl.BlockSpec((1, _H), lambda b: (0, 0)),
        ],
        out_specs=pl.BlockSpec((1, _H, N + 1, N + 1), lambda b: (b, 0, 0, 0)),
        out_shape=jax.ShapeDtypeStruct((B, _H, N + 1, N + 1), jnp.float32),
    )(interim, attn_bias, vdist_w)


def kernel(attn_bias, spatial_pos, edge_input, attn_edge_type,
           edge_enc_w, spatial_enc_w, edge_dis_w, vdist_w):
    del attn_edge_type
    B, N = spatial_pos.shape[:2]

    dis5 = edge_dis_w.reshape(-1, _H, _H)[:_MAX_DIST]
    ct = _build_tables(edge_enc_w, spatial_enc_w, dis5)
    ct = ct.reshape(_NQ, _CT_ROWS * _CTS)

    sp = spatial_pos.astype(jnp.int32).reshape(B * N, N)
    ei = edge_input.astype(jnp.int32).reshape(B * N, N * 15)

    interim = _sc_interior(ct, sp, ei, B, N)
    return _assemble(interim, attn_bias, vdist_w, B, N)
